# baseline (speedup 1.0000x reference)
import functools

import jax
import jax.numpy as jnp
from jax import lax
from jax.experimental import pallas as pl
from jax.experimental.pallas import tpu as pltpu
from jax.experimental.pallas import tpu_sc as plsc

VOCAB = 1000000
EMB = 32
BATCH = 16384
HIST = 50

N = BATCH * HIST
NC, NS = 2, 16
NW = NC * NS
B_PER_W = N // NW
CHUNK = 1600
NCHUNK = B_PER_W // CHUNK
NBUF = 2

_mesh = plsc.VectorSubcoreMesh(core_axis_name="c", subcore_axis_name="s")


@functools.partial(
    pl.kernel,
    out_type=jax.ShapeDtypeStruct((N, EMB), jnp.float32),
    mesh=_mesh,
    scratch_types=[
        pltpu.VMEM((NBUF, CHUNK), jnp.int32),
        pltpu.VMEM((NBUF, CHUNK, EMB), jnp.float32),
        [pltpu.SemaphoreType.DMA] * NBUF,
        [pltpu.SemaphoreType.DMA] * NBUF,
    ],
    compiler_params=pltpu.CompilerParams(use_tc_tiling_on_sc=False),
)
def _gather_kernel(idx_hbm, table_hbm, out_hbm, idx_v, rows_v, gsem, wsem):
    wid = lax.axis_index("s") * NC + lax.axis_index("c")
    w_base = wid * B_PER_W

    gathers = [None] * NBUF
    writes = [None] * NBUF

    def start_gather(i, b):
        base = w_base + i * CHUNK
        pltpu.sync_copy(idx_hbm.at[pl.ds(base, CHUNK)], idx_v.at[b])
        gathers[b] = pltpu.async_copy(table_hbm.at[idx_v.at[b]],
                                      rows_v.at[b], gsem[b])

    for b in range(NBUF):
        start_gather(b, b)
    for i in range(NCHUNK):
        b = i % NBUF
        gathers[b].wait()
        writes[b] = pltpu.async_copy(
            rows_v.at[b], out_hbm.at[pl.ds(w_base + i * CHUNK, CHUNK)],
            wsem[b])
        j = i + NBUF
        if j < NCHUNK:
            writes[b].wait()
            start_gather(j, b)
    for b in range(NBUF):
        writes[b].wait()


def kernel(x, table):
    idx = x.reshape(N).astype(jnp.int32)
    out = _gather_kernel(idx, table)
    return out.reshape(BATCH, HIST, EMB)


# speedup vs baseline: 1.7290x; 1.7290x over previous
import functools

import jax
import jax.numpy as jnp
from jax import lax
from jax.experimental import pallas as pl
from jax.experimental.pallas import tpu as pltpu
from jax.experimental.pallas import tpu_sc as plsc

VOCAB = 1000000
EMB = 32
BATCH = 16384
HIST = 50

NC, NS = 2, 16
NW = NC * NS
COLS = BATCH // NW
RH = 2
NSTEP = HIST // RH
NBUF = 2

_mesh = plsc.VectorSubcoreMesh(core_axis_name="c", subcore_axis_name="s")


@functools.partial(
    pl.kernel,
    out_type=jax.ShapeDtypeStruct((HIST, BATCH, EMB), jnp.float32),
    mesh=_mesh,
    scratch_types=[
        pltpu.VMEM((NBUF, RH, COLS), jnp.int32),
        pltpu.VMEM((NBUF, RH, COLS, EMB), jnp.float32),
        [pltpu.SemaphoreType.DMA] * NBUF,
        [pltpu.SemaphoreType.DMA] * NBUF,
    ],
    compiler_params=pltpu.CompilerParams(use_tc_tiling_on_sc=False),
)
def _gather_kernel(xt_hbm, table_hbm, out_hbm, idx_v, rows_v, gsem, wsem):
    wid = lax.axis_index("s") * NC + lax.axis_index("c")
    c0 = wid * COLS

    gathers = [None] * NBUF
    writes = [None] * NBUF

    def start_gather(i, b):
        pltpu.sync_copy(xt_hbm.at[pl.ds(i * RH, RH), pl.ds(c0, COLS)],
                        idx_v.at[b])
        gathers[b] = [
            pltpu.async_copy(table_hbm.at[idx_v.at[b, r]], rows_v.at[b, r],
                             gsem[b])
            for r in range(RH)
        ]

    for b in range(NBUF):
        start_gather(b, b)
    for i in range(NSTEP):
        b = i % NBUF
        for g in gathers[b]:
            g.wait()
        writes[b] = pltpu.async_copy(
            rows_v.at[b],
            out_hbm.at[pl.ds(i * RH, RH), pl.ds(c0, COLS), :],
            wsem[b])
        j = i + NBUF
        if j < NSTEP:
            writes[b].wait()
            start_gather(j, b)
    for b in range(NBUF):
        writes[b].wait()


def kernel(x, table):
    out = _gather_kernel(x.T.astype(jnp.int32), table)
    return out.transpose(1, 0, 2)
